# baseline (speedup 1.0000x reference)
import functools

import jax
import jax.numpy as jnp
from jax import lax
from jax.experimental import pallas as pl
from jax.experimental.pallas import tpu as pltpu
from jax.experimental.pallas import tpu_sc as plsc

VOCAB_SIZE = 100000
P = 16
ALPHA = 8
D = P * ALPHA
B = 1024
L = 200

NC = 2
NS = 16
NW = NC * NS

N_ROWS = B * L
ROWS_PER_W = N_ROWS // NW
GCHUNK = 128
GPC = 2
CHUNK = GCHUNK * GPC
N_CHUNKS = ROWS_PER_W // CHUNK
NBUF = 3


@functools.partial(
    pl.kernel,
    out_type=jax.ShapeDtypeStruct((N_ROWS, D), jnp.float32),
    mesh=plsc.VectorSubcoreMesh(core_axis_name="c", subcore_axis_name="s",
                                num_cores=NC, num_subcores=NS),
    scratch_types=[
        pltpu.VMEM((N_CHUNKS * GPC, GCHUNK), jnp.int32),
        [pltpu.VMEM((CHUNK, D), jnp.float32) for _ in range(NBUF)],
        [pltpu.SemaphoreType.DMA for _ in range(NBUF)],
        [pltpu.SemaphoreType.DMA for _ in range(NBUF)],
    ],
)
def _gather_kernel(idx_hbm, table_hbm, out_hbm, idx_v, bufs, gsems, osems):
    wid = lax.axis_index("s") * NC + lax.axis_index("c")
    base = wid * ROWS_PER_W
    pltpu.sync_copy(idx_hbm.at[wid], idx_v)

    def start_gathers(c, pos):
        for h in range(GPC):
            pltpu.async_copy(table_hbm.at[idx_v.at[c * GPC + h]],
                             bufs[pos].at[pl.ds(h * GCHUNK, GCHUNK)],
                             gsems[pos])

    def wait_gathers(c, pos):
        for h in range(GPC):
            pltpu.make_async_copy(table_hbm.at[idx_v.at[c * GPC + h]],
                                  bufs[pos].at[pl.ds(h * GCHUNK, GCHUNK)],
                                  gsems[pos]).wait()

    def start_out(c, pos):
        pltpu.async_copy(bufs[pos], out_hbm.at[pl.ds(base + c * CHUNK, CHUNK)],
                         osems[pos])

    def wait_out(c, pos):
        pltpu.make_async_copy(bufs[pos],
                              out_hbm.at[pl.ds(base + c * CHUNK, CHUNK)],
                              osems[pos]).wait()

    def process(c, cpos):
        pos = cpos % NBUF
        wait_gathers(c, pos)
        start_out(c, pos)
        if cpos + 2 < N_CHUNKS:
            npos = (cpos + 2) % NBUF
            if cpos + 2 >= NBUF:
                wait_out(c - 1, npos)
            start_gathers(c + 2, npos)

    start_gathers(0, 0)
    start_gathers(1, 1)

    process(0, 0)
    process(1, 1)

    def body(i, carry):
        cb = 2 + i * NBUF
        for off in range(NBUF):
            process(cb + off, 2 + off)
        return carry

    lax.fori_loop(0, (N_CHUNKS - 4) // NBUF, body, 0)

    process(23, 23)
    process(24, 24)

    for c in range(N_CHUNKS - NBUF, N_CHUNKS):
        wait_out(c, c % NBUF)


def kernel(input_ids, embedding):
    idx = input_ids.reshape(NW, N_CHUNKS * GPC, GCHUNK).astype(jnp.int32)
    out = _gather_kernel(idx, embedding)
    return out.reshape(B, L, D)


# speedup vs baseline: 1.0137x; 1.0137x over previous
import functools

import jax
import jax.numpy as jnp
from jax import lax
from jax.experimental import pallas as pl
from jax.experimental.pallas import tpu as pltpu
from jax.experimental.pallas import tpu_sc as plsc

VOCAB_SIZE = 100000
P = 16
ALPHA = 8
D = P * ALPHA
B = 1024
L = 200

NC = 2
NS = 16
NW = NC * NS

N_ROWS = B * L
ROWS_PER_W = N_ROWS // NW
CHUNK = 128
N_CHUNKS = ROWS_PER_W // CHUNK
NBUF = 6
PF = 3

_STEADY = N_CHUNKS - NBUF
_ITERS = _STEADY // NBUF
HEAD_END = (NBUF - PF) + (_STEADY - _ITERS * NBUF)
TAIL_LO = HEAD_END + _ITERS * NBUF


@functools.partial(
    pl.kernel,
    out_type=jax.ShapeDtypeStruct((N_ROWS, D), jnp.float32),
    mesh=plsc.VectorSubcoreMesh(core_axis_name="c", subcore_axis_name="s",
                                num_cores=NC, num_subcores=NS),
    scratch_types=[
        pltpu.VMEM((ROWS_PER_W,), jnp.int32),
        [pltpu.VMEM((CHUNK, D), jnp.float32) for _ in range(NBUF)],
        [pltpu.SemaphoreType.DMA for _ in range(NBUF)],
        [pltpu.SemaphoreType.DMA for _ in range(NBUF)],
    ],
)
def _gather_kernel(idx_hbm, table_hbm, out_hbm, idx_v, bufs, gsems, osems):
    wid = lax.axis_index("s") * NC + lax.axis_index("c")
    base = wid * ROWS_PER_W
    pltpu.sync_copy(idx_hbm.at[pl.ds(base, ROWS_PER_W)], idx_v)

    def start_gather(j, pos):
        pltpu.async_copy(table_hbm.at[idx_v.at[pl.ds(j * CHUNK, CHUNK)]], bufs[pos], gsems[pos])

    def wait_gather(j, pos):
        pltpu.make_async_copy(table_hbm.at[idx_v.at[pl.ds(j * CHUNK, CHUNK)]], bufs[pos],
                              gsems[pos]).wait()

    def start_out(j, pos):
        pltpu.async_copy(bufs[pos], out_hbm.at[pl.ds(base + j * CHUNK, CHUNK)],
                         osems[pos])

    def wait_out(j, pos):
        pltpu.make_async_copy(bufs[pos],
                              out_hbm.at[pl.ds(base + j * CHUNK, CHUNK)],
                              osems[pos]).wait()

    def process(j, jpos):
        pos = jpos % NBUF
        wait_gather(j, pos)
        start_out(j, pos)
        if jpos + PF < N_CHUNKS:
            npos = (jpos + PF) % NBUF
            if jpos + PF >= NBUF:
                wait_out(j + PF - NBUF, npos)
            start_gather(j + PF, npos)

    for j in range(PF):
        start_gather(j, j % NBUF)

    for j in range(HEAD_END):
        process(j, j)

    def body(i, carry):
        jb = HEAD_END + i * NBUF
        for off in range(NBUF):
            process(jb + off, HEAD_END + off)
        return carry

    lax.fori_loop(0, _ITERS, body, 0)

    for j in range(TAIL_LO, N_CHUNKS):
        process(j, j)

    for j in range(N_CHUNKS - NBUF, N_CHUNKS):
        wait_out(j, j % NBUF)


def kernel(input_ids, embedding):
    idx = input_ids.reshape(N_ROWS).astype(jnp.int32)
    out = _gather_kernel(idx, embedding)
    return out.reshape(B, L, D)


# speedup vs baseline: 1.0138x; 1.0001x over previous
import functools

import jax
import jax.numpy as jnp
from jax import lax
from jax.experimental import pallas as pl
from jax.experimental.pallas import tpu as pltpu
from jax.experimental.pallas import tpu_sc as plsc

VOCAB_SIZE = 100000
P = 16
ALPHA = 8
D = P * ALPHA
B = 1024
L = 200

NC = 2
NS = 16
NW = NC * NS

N_ROWS = B * L
ROWS_PER_W = N_ROWS // NW
CHUNK = 128
N_CHUNKS = ROWS_PER_W // CHUNK
NBUF = 6
PF = 4

_STEADY = N_CHUNKS - NBUF
_ITERS = _STEADY // NBUF
HEAD_END = (NBUF - PF) + (_STEADY - _ITERS * NBUF)
TAIL_LO = HEAD_END + _ITERS * NBUF


@functools.partial(
    pl.kernel,
    out_type=jax.ShapeDtypeStruct((N_ROWS, D), jnp.float32),
    mesh=plsc.VectorSubcoreMesh(core_axis_name="c", subcore_axis_name="s",
                                num_cores=NC, num_subcores=NS),
    scratch_types=[
        pltpu.VMEM((ROWS_PER_W,), jnp.int32),
        [pltpu.VMEM((CHUNK, D), jnp.float32) for _ in range(NBUF)],
        [pltpu.SemaphoreType.DMA for _ in range(NBUF)],
        [pltpu.SemaphoreType.DMA for _ in range(NBUF)],
    ],
)
def _gather_kernel(idx_hbm, table_hbm, out_hbm, idx_v, bufs, gsems, osems):
    wid = lax.axis_index("s") * NC + lax.axis_index("c")
    base = wid * ROWS_PER_W
    pltpu.sync_copy(idx_hbm.at[pl.ds(base, ROWS_PER_W)], idx_v)

    def start_gather(j, pos):
        pltpu.async_copy(table_hbm.at[idx_v.at[pl.ds(j * CHUNK, CHUNK)]], bufs[pos], gsems[pos])

    def wait_gather(j, pos):
        pltpu.make_async_copy(table_hbm.at[idx_v.at[pl.ds(j * CHUNK, CHUNK)]], bufs[pos],
                              gsems[pos]).wait()

    def start_out(j, pos):
        pltpu.async_copy(bufs[pos], out_hbm.at[pl.ds(base + j * CHUNK, CHUNK)],
                         osems[pos])

    def wait_out(j, pos):
        pltpu.make_async_copy(bufs[pos],
                              out_hbm.at[pl.ds(base + j * CHUNK, CHUNK)],
                              osems[pos]).wait()

    def process(j, jpos):
        pos = jpos % NBUF
        wait_gather(j, pos)
        start_out(j, pos)
        if jpos + PF < N_CHUNKS:
            npos = (jpos + PF) % NBUF
            if jpos + PF >= NBUF:
                wait_out(j + PF - NBUF, npos)
            start_gather(j + PF, npos)

    for j in range(PF):
        start_gather(j, j % NBUF)

    for j in range(HEAD_END):
        process(j, j)

    def body(i, carry):
        jb = HEAD_END + i * NBUF
        for off in range(NBUF):
            process(jb + off, HEAD_END + off)
        return carry

    lax.fori_loop(0, _ITERS, body, 0)

    for j in range(TAIL_LO, N_CHUNKS):
        process(j, j)

    for j in range(N_CHUNKS - NBUF, N_CHUNKS):
        wait_out(j, j % NBUF)


def kernel(input_ids, embedding):
    idx = input_ids.reshape(N_ROWS).astype(jnp.int32)
    out = _gather_kernel(idx, embedding)
    return out.reshape(B, L, D)
